# baseline (speedup 1.0000x reference)
import jax
import jax.numpy as jnp
from jax import lax
from jax.experimental import pallas as pl
from jax.experimental.pallas import tpu as pltpu
from jax.experimental.pallas import tpu_sc as plsc

NC = 2
NS = 16
NW = NC * NS
L = 16
BB = 128
NB = 4


def _emb_body(tok_hbm, table_hbm, out_hbm, idx_v, rows, tsems, gsems, wsems):
    D = table_hbm.shape[1]
    scale = float(D) ** 0.5
    n_items = tok_hbm.shape[0] // BB
    ipw = n_items // NW
    wid = lax.axis_index("s") * NC + lax.axis_index("c")
    m0 = wid * ipw

    def tok_slice(m):
        return tok_hbm.at[pl.ds(m * BB, BB)]

    def tok_start(m, s):
        pltpu.async_copy(tok_slice(m), idx_v.at[s], tsems[s])

    def tok_wait(s):
        pltpu.make_async_copy(tok_slice(0), idx_v.at[s], tsems[s]).wait()

    def gather_start(b, s):
        pltpu.async_copy(table_hbm.at[idx_v.at[s]], rows[b], gsems[b])

    def gather_wait(b, s):
        pltpu.make_async_copy(table_hbm.at[idx_v.at[s]], rows[b],
                              gsems[b]).wait()

    def out_slice(m):
        return out_hbm.at[pl.ds(m * BB, BB)]

    def write_start(m, b):
        pltpu.async_copy(rows[b], out_slice(m), wsems[b])

    def write_drain(b):
        pltpu.make_async_copy(rows[b], out_slice(0), wsems[b]).wait()

    for j in range(3):
        tok_start(m0 + j, j)
    tok_wait(0)
    gather_start(0, 0)

    @pl.loop(0, ipw, step=NB)
    def block(k0):
        for j in range(NB):
            m = k0 + j
            b = j % NB
            s = j % 4

            @pl.when(m + 3 < ipw)
            def _():
                tok_start(m0 + m + 3, (j + 3) % 4)

            @pl.when(m + 1 < ipw)
            def _():
                tok_wait((j + 1) % 4)

                @pl.when(m >= NB - 1)
                def _():
                    write_drain((j + 1) % NB)

                gather_start((j + 1) % NB, (j + 1) % 4)

            gather_wait(b, s)

            @plsc.parallel_loop(0, BB, step=1, unroll=4)
            def scale_row(r):
                for c in range(D // L):
                    v = rows[b][r, pl.ds(c * L, L)]
                    rows[b][r, pl.ds(c * L, L)] = v * scale

            write_start(m0 + m, b)

    for b in range(NB):
        write_drain(b)


def kernel(tokens, table):
    Bdim, T = tokens.shape
    V, D = table.shape
    tok_flat = tokens.reshape(-1).astype(jnp.int32)
    mesh = plsc.VectorSubcoreMesh(
        core_axis_name="c", subcore_axis_name="s",
        num_cores=NC, num_subcores=NS,
    )
    out2 = pl.kernel(
        _emb_body,
        out_type=jax.ShapeDtypeStruct((Bdim * T, D), table.dtype),
        mesh=mesh,
        scratch_types=[
            pltpu.VMEM((4, BB), jnp.int32),
            [pltpu.VMEM((BB, D), jnp.float32) for _ in range(NB)],
            [pltpu.SemaphoreType.DMA for _ in range(4)],
            [pltpu.SemaphoreType.DMA for _ in range(NB)],
            [pltpu.SemaphoreType.DMA for _ in range(NB)],
        ],
        compiler_params=pltpu.CompilerParams(use_tc_tiling_on_sc=False),
    )(tok_flat, table)
    return out2.reshape(Bdim, T, D)


# speedup vs baseline: 1.5906x; 1.5906x over previous
import jax
import jax.numpy as jnp
from jax import lax
from jax.experimental import pallas as pl
from jax.experimental.pallas import tpu as pltpu
from jax.experimental.pallas import tpu_sc as plsc

NC = 2
NS = 16
NW = NC * NS
L = 16
BB = 128
XP = 129


def _emb_body(tokt_hbm, table_hbm, out_hbm, idx_v, rows, xps,
              tsems, gsems, wsems):
    D = table_hbm.shape[1]
    scale = float(D) ** 0.5
    ncg = D // 8
    nbb = out_hbm.shape[2]
    n_items = tokt_hbm.shape[0] // BB
    ipw = n_items // NW
    wid = lax.axis_index("s") * NC + lax.axis_index("c")
    m0 = wid * ipw

    col_ids = [lax.iota(jnp.int32, L) + c * L for c in range(D // L)]

    def tok_slice(m):
        return tokt_hbm.at[pl.ds(m * BB, BB)]

    def tok_start(m, s):
        pltpu.async_copy(tok_slice(m), idx_v.at[s], tsems[s])

    def tok_wait(s):
        pltpu.make_async_copy(tok_slice(0), idx_v.at[s], tsems[s]).wait()

    def gather_start(b, s):
        pltpu.async_copy(table_hbm.at[idx_v.at[s]], rows[b], gsems[b])

    def gather_wait(b, s):
        pltpu.make_async_copy(table_hbm.at[idx_v.at[s]], rows[b],
                              gsems[b]).wait()

    def write_start(m, x):
        t = m // nbb
        bb = m % nbb
        for cg in range(ncg):
            pltpu.async_copy(xps[x].at[pl.ds(cg * 8, 8), pl.ds(0, BB)],
                             out_hbm.at[t, cg, bb], wsems[x])

    def write_drain(x):
        for cg in range(ncg):
            pltpu.make_async_copy(xps[x].at[pl.ds(cg * 8, 8), pl.ds(0, BB)],
                                  out_hbm.at[0, cg, 0], wsems[x]).wait()

    for j in range(3):
        tok_start(m0 + j, j)
    tok_wait(0)
    gather_start(0, 0)

    @pl.loop(0, ipw, step=4)
    def block(k0):
        for j in range(4):
            m = k0 + j
            b = j % 2
            x = j % 2
            s = j % 4

            @pl.when(m + 3 < ipw)
            def _():
                tok_start(m0 + m + 3, (j + 3) % 4)

            @pl.when(m + 1 < ipw)
            def _():
                tok_wait((j + 1) % 4)
                gather_start((j + 1) % 2, (j + 1) % 4)

            gather_wait(b, s)

            @pl.when(m >= 2)
            def _():
                write_drain(x)

            @plsc.parallel_loop(0, BB, step=1, unroll=4)
            def transpose_row(r):
                rv = jnp.full((L,), r, dtype=jnp.int32)
                for c in range(D // L):
                    v = rows[b][r, pl.ds(c * L, L)]
                    plsc.store_scatter(xps[x], [col_ids[c], rv], v * scale)

            write_start(m0 + m, x)

    for x in range(2):
        write_drain(x)


def kernel(tokens, table):
    Bdim, T = tokens.shape
    V, D = table.shape
    nbb = Bdim // BB
    tokt = jnp.transpose(tokens).reshape(-1).astype(jnp.int32)
    mesh = plsc.VectorSubcoreMesh(
        core_axis_name="c", subcore_axis_name="s",
        num_cores=NC, num_subcores=NS,
    )
    out5 = pl.kernel(
        _emb_body,
        out_type=jax.ShapeDtypeStruct((T, D // 8, nbb, 8, BB), table.dtype),
        mesh=mesh,
        scratch_types=[
            pltpu.VMEM((4, BB), jnp.int32),
            [pltpu.VMEM((BB, D), jnp.float32) for _ in range(2)],
            [pltpu.VMEM((D, XP), jnp.float32) for _ in range(2)],
            [pltpu.SemaphoreType.DMA for _ in range(4)],
            [pltpu.SemaphoreType.DMA for _ in range(2)],
            [pltpu.SemaphoreType.DMA for _ in range(2)],
        ],
        compiler_params=pltpu.CompilerParams(
            use_tc_tiling_on_sc=False, needs_layout_passes=False),
    )(tokt, table)
    return out5.transpose(2, 4, 0, 1, 3).reshape(Bdim, T, D)
